# baseline (speedup 1.0000x reference)
import functools

import jax
import jax.numpy as jnp
from jax import lax
from jax.experimental import pallas as pl
from jax.experimental.pallas import tpu as pltpu
from jax.experimental.pallas import tpu_sc as plsc

D = 128
WIDTH = 144
NC = 2
NS = 16
NW = NC * NS
CHUNK = 64
NBUF = 2
KBLK = 8


def _sc_agg_body(nodes_pad, nchunk,
                 feats, srcp, dstp, wp, zrows, out,
                 gath0, gath1, scat0, scat1, sidx, didx, wbuf, table,
                 gsem0, gsem1, ssem0, ssem1):
    cid = lax.axis_index("c")
    sid = lax.axis_index("s")
    wid = sid * NC + cid
    rows_per_tile = nodes_pad // NS
    r0 = sid * rows_per_tile
    bufs = ((gath0, scat0, gsem0, ssem0), (gath1, scat1, gsem1, ssem1))

    pltpu.sync_copy(srcp.at[wid, pl.ds(0, KBLK)], sidx.at[pl.ds(0, KBLK)])
    pltpu.sync_copy(dstp.at[wid, pl.ds(0, KBLK)], didx.at[pl.ds(0, KBLK)])
    pltpu.sync_copy(wp.at[wid, pl.ds(0, KBLK)], wbuf.at[pl.ds(0, KBLK)])

    pltpu.sync_copy(zrows, table.at[pl.ds(r0, rows_per_tile)])

    ecol = jnp.where(lax.iota(jnp.int32, 16) == 0,
                     jnp.float32(1.0), jnp.float32(0.0))

    @pl.loop(0, CHUNK)
    def _init_count_col(e):
        scat0[e, pl.ds(D, 16)] = ecol
        scat1[e, pl.ds(D, 16)] = ecol

    pltpu.async_copy(feats.at[sidx.at[0]], gath0, gsem0)
    pltpu.async_copy(feats.at[sidx.at[1]], gath1, gsem1)

    plsc.subcore_barrier()

    ring = 2 * KBLK

    @pl.loop(0, nchunk, step=NBUF)
    def _pair(g0):
        for b, (gath, scat, gsem, ssem) in enumerate(bufs):
            g = g0 + b
            row = lax.rem(g, ring)
            pltpu.make_async_copy(feats.at[sidx.at[row]], gath, gsem).wait()
            @pl.when(g0 > 0)
            def _():
                pltpu.make_async_copy(scat, table.at[didx.at[row]], ssem).wait()

            @pl.loop(0, CHUNK // 16, unroll=2)
            def _scale(grp):
                wv16 = wbuf[row, pl.ds(grp * 16, 16)]
                for k in range(16):
                    ws = wv16[k]
                    e = grp * 16 + k
                    for j in range(D // 16):
                        scat[e, pl.ds(j * 16, 16)] = (
                            gath[e, pl.ds(j * 16, 16)] * ws)

            pltpu.make_async_copy(scat, table.at[didx.at[row]],
                                  ssem).start(add=True)

            @pl.when(g + NBUF < nchunk)
            def _():
                pltpu.async_copy(feats.at[sidx.at[lax.rem(g + NBUF, ring)]],
                                 gath, gsem)

        @pl.when(lax.rem(g0, KBLK) == KBLK - 4)
        def _():
            nb = g0 // KBLK + 1

            @pl.when(nb < nchunk // KBLK)
            def _():
                half = lax.rem(nb, 2) * KBLK
                pltpu.sync_copy(srcp.at[wid, pl.ds(nb * KBLK, KBLK)],
                                sidx.at[pl.ds(half, KBLK)])
                pltpu.sync_copy(dstp.at[wid, pl.ds(nb * KBLK, KBLK)],
                                didx.at[pl.ds(half, KBLK)])
                pltpu.sync_copy(wp.at[wid, pl.ds(nb * KBLK, KBLK)],
                                wbuf.at[pl.ds(half, KBLK)])

    pltpu.make_async_copy(scat0, table.at[didx.at[0]], ssem0).wait()
    pltpu.make_async_copy(scat1, table.at[didx.at[1]], ssem1).wait()

    plsc.subcore_barrier()
    pltpu.sync_copy(table.at[pl.ds(r0, rows_per_tile)],
                    out.at[cid, pl.ds(r0, rows_per_tile)])


def _sc_agg(feats, srcp, dstp, wp, zrows, nodes_pad):
    nchunk = srcp.shape[1]
    mesh = plsc.VectorSubcoreMesh(core_axis_name="c", subcore_axis_name="s")
    body = functools.partial(_sc_agg_body, nodes_pad, nchunk)
    return pl.kernel(
        body,
        out_type=jax.ShapeDtypeStruct((NC, nodes_pad, WIDTH), jnp.float32),
        mesh=mesh,
        compiler_params=pltpu.CompilerParams(use_tc_tiling_on_sc=False),
        scratch_types=[
            pltpu.VMEM((CHUNK, D), jnp.float32),
            pltpu.VMEM((CHUNK, D), jnp.float32),
            pltpu.VMEM((CHUNK, WIDTH), jnp.float32),
            pltpu.VMEM((CHUNK, WIDTH), jnp.float32),
            pltpu.VMEM((2 * KBLK, CHUNK), jnp.int32),
            pltpu.VMEM((2 * KBLK, CHUNK), jnp.int32),
            pltpu.VMEM((2 * KBLK, CHUNK), jnp.float32),
            pltpu.VMEM_SHARED((nodes_pad, WIDTH), jnp.float32),
            pltpu.SemaphoreType.DMA,
            pltpu.SemaphoreType.DMA,
            pltpu.SemaphoreType.DMA,
            pltpu.SemaphoreType.DMA,
        ],
    )(feats, srcp, dstp, wp, zrows)


def _tc_body(relu, p_ref, x_ref, wl_ref, wr_ref, b_ref, o_ref):
    p = p_ref[...]
    agg = p[0, :, :D] + p[1, :, :D]
    cnt = p[0, :, D:D + 1] + p[1, :, D:D + 1]
    inv = 1.0 / jnp.maximum(cnt, 1.0)
    y = jnp.dot(agg * inv, wl_ref[...], preferred_element_type=jnp.float32)
    y = y + jnp.dot(x_ref[...], wr_ref[...], preferred_element_type=jnp.float32)
    y = y + b_ref[...]
    if relu:
        y = jnp.maximum(y, 0.0)
    o_ref[...] = y


def _tc_sage(p, x, wl, wr, b, relu):
    n = x.shape[0]
    blk = 1000
    grid = n // blk
    return pl.pallas_call(
        functools.partial(_tc_body, relu),
        grid=(grid,),
        in_specs=[
            pl.BlockSpec((NC, blk, WIDTH), lambda i: (0, i, 0)),
            pl.BlockSpec((blk, D), lambda i: (i, 0)),
            pl.BlockSpec((D, D), lambda i: (0, 0)),
            pl.BlockSpec((D, D), lambda i: (0, 0)),
            pl.BlockSpec((1, D), lambda i: (0, 0)),
        ],
        out_specs=pl.BlockSpec((blk, D), lambda i: (i, 0)),
        out_shape=jax.ShapeDtypeStruct((n, D), jnp.float32),
    )(p, x, wl, wr, b)


def kernel(x, edge_index, edge_weight, node_type, Wl1, Wr1, b1, Wl2, Wr2, b2):
    del node_type
    n = x.shape[0]
    e = edge_weight.shape[0]
    nodes_pad = ((n + 1 + NS * 8 - 1) // (NS * 8)) * (NS * 8)
    grain = NW * CHUNK * KBLK
    ep = ((e + grain - 1) // grain) * grain

    src = edge_index[0].astype(jnp.int32)
    dst = edge_index[1].astype(jnp.int32)
    w = edge_weight.astype(jnp.float32)
    perm = jnp.argsort(src)
    src = src[perm]
    dst = dst[perm]
    w = w[perm]
    pad = ep - e
    srcp = jnp.concatenate([src, jnp.zeros((pad,), jnp.int32)])
    dstp = jnp.concatenate([dst, jnp.full((pad,), n, jnp.int32)])
    wp = jnp.concatenate([w, jnp.zeros((pad,), jnp.float32)])
    nchunk = ep // (NW * CHUNK)
    srcp = srcp.reshape(NW, nchunk, CHUNK)
    dstp = dstp.reshape(NW, nchunk, CHUNK)
    wp = wp.reshape(NW, nchunk, CHUNK)
    zrows = jnp.zeros((nodes_pad // NS, WIDTH), jnp.float32)
    b1r = b1.reshape(1, D)
    b2r = b2.reshape(1, D)

    p = _sc_agg(x, srcp, dstp, wp, zrows, nodes_pad)
    h = _tc_sage(p, x, Wl1, Wr1, b1r, relu=True)
    q = _sc_agg(h, srcp, dstp, wp, zrows, nodes_pad)
    out = _tc_sage(q, h, Wl2, Wr2, b2r, relu=False)
    return out


# speedup vs baseline: 2.1521x; 2.1521x over previous
import functools

import jax
import jax.numpy as jnp
import numpy as np
from jax import lax
from jax.experimental import pallas as pl
from jax.experimental.pallas import tpu as pltpu
from jax.experimental.pallas import tpu_sc as plsc

D = 128
WIDTH = 144
NC = 2
NS = 16
NW = NC * NS
CHUNK = 64
NBUF = 4
KBLK = 8

_PERM = np.concatenate(
    [np.concatenate([np.arange(j * 32, (j + 1) * 32, 2),
                     np.arange(j * 32 + 1, (j + 1) * 32, 2)])
     for j in range(D // 32)])


def _sc_agg_body(nodes_pad, nchunk,
                 feats, srcp, dstp, wp, zrows, out,
                 gath0, gath1, gath2, gath3, scat0, scat1,
                 sidx, didx, wbuf, table,
                 gsem0, gsem1, gsem2, gsem3, ssem0, ssem1):
    cid = lax.axis_index("c")
    sid = lax.axis_index("s")
    wid = sid * NC + cid
    rows_per_tile = nodes_pad // NS
    r0 = sid * rows_per_tile
    gaths = (gath0, gath1, gath2, gath3)
    gsems = (gsem0, gsem1, gsem2, gsem3)
    scats = (scat0, scat1)
    ssems = (ssem0, ssem1)

    pltpu.sync_copy(srcp.at[wid, pl.ds(0, KBLK)], sidx.at[pl.ds(0, KBLK)])
    pltpu.sync_copy(dstp.at[wid, pl.ds(0, KBLK)], didx.at[pl.ds(0, KBLK)])
    pltpu.sync_copy(wp.at[wid, pl.ds(0, KBLK)], wbuf.at[pl.ds(0, KBLK)])

    pltpu.sync_copy(zrows, table.at[pl.ds(r0, rows_per_tile)])

    ecol = jnp.where(lax.iota(jnp.int32, 16) == 0,
                     jnp.float32(1.0), jnp.float32(0.0))

    @pl.loop(0, CHUNK)
    def _init_count_col(e):
        scat0[e, pl.ds(D, 16)] = ecol
        scat1[e, pl.ds(D, 16)] = ecol

    pltpu.async_copy(feats.at[sidx.at[0]], gath0, gsem0)
    pltpu.async_copy(feats.at[sidx.at[1]], gath1, gsem1)
    pltpu.async_copy(feats.at[sidx.at[2]], gath2, gsem2)

    plsc.subcore_barrier()

    ring = 2 * KBLK
    himask = jnp.int32(-65536)

    @pl.loop(0, nchunk, step=NBUF)
    def _quad(g0):
        for b in range(NBUF):
            gath, gsem = gaths[b], gsems[b]
            scat, ssem = scats[b % 2], ssems[b % 2]
            g = g0 + b
            row = lax.rem(g, ring)
            pltpu.make_async_copy(feats.at[sidx.at[row]], gath, gsem).wait()

            @pl.when(g + NBUF - 1 < nchunk)
            def _():
                nxt = lax.rem(g + NBUF - 1, ring)
                pltpu.async_copy(feats.at[sidx.at[nxt]],
                                 gaths[(b + NBUF - 1) % NBUF],
                                 gsems[(b + NBUF - 1) % NBUF])

            if b < 2:
                @pl.when(g0 > 0)
                def _():
                    pltpu.make_async_copy(
                        scat, table.at[didx.at[row]], ssem).wait()
            else:
                pltpu.make_async_copy(
                    scat, table.at[didx.at[row]], ssem).wait()

            @pl.loop(0, CHUNK // 16)
            def _scale(grp):
                wv16 = wbuf[row, pl.ds(grp * 16, 16)]
                for k in range(16):
                    ws = wv16[k]
                    e = grp * 16 + k
                    for j in range(D // 32):
                        v = gath[e, pl.ds(j * 16, 16)]
                        lo = plsc.bitcast(v << 16, jnp.float32)
                        hi = plsc.bitcast(v & himask, jnp.float32)
                        scat[e, pl.ds(j * 32, 16)] = lo * ws
                        scat[e, pl.ds(j * 32 + 16, 16)] = hi * ws

            pltpu.make_async_copy(scat, table.at[didx.at[row]],
                                  ssem).start(add=True)

        @pl.when(lax.rem(g0, KBLK) == 0)
        def _():
            nb = g0 // KBLK + 1

            @pl.when(nb < nchunk // KBLK)
            def _():
                half = lax.rem(nb, 2) * KBLK
                pltpu.sync_copy(srcp.at[wid, pl.ds(nb * KBLK, KBLK)],
                                sidx.at[pl.ds(half, KBLK)])
                pltpu.sync_copy(dstp.at[wid, pl.ds(nb * KBLK, KBLK)],
                                didx.at[pl.ds(half, KBLK)])
                pltpu.sync_copy(wp.at[wid, pl.ds(nb * KBLK, KBLK)],
                                wbuf.at[pl.ds(half, KBLK)])

    pltpu.make_async_copy(scat0, table.at[didx.at[0]], ssem0).wait()
    pltpu.make_async_copy(scat1, table.at[didx.at[1]], ssem1).wait()

    plsc.subcore_barrier()
    pltpu.sync_copy(table.at[pl.ds(r0, rows_per_tile)],
                    out.at[cid, pl.ds(r0, rows_per_tile)])


def _sc_agg(feats, srcp, dstp, wp, zrows, nodes_pad):
    nchunk = srcp.shape[1]
    mesh = plsc.VectorSubcoreMesh(core_axis_name="c", subcore_axis_name="s")
    body = functools.partial(_sc_agg_body, nodes_pad, nchunk)
    return pl.kernel(
        body,
        out_type=jax.ShapeDtypeStruct((NC, nodes_pad, WIDTH), jnp.float32),
        mesh=mesh,
        compiler_params=pltpu.CompilerParams(use_tc_tiling_on_sc=False,
                                             needs_layout_passes=False),
        scratch_types=[
            pltpu.VMEM((CHUNK, D // 2), jnp.int32),
            pltpu.VMEM((CHUNK, D // 2), jnp.int32),
            pltpu.VMEM((CHUNK, D // 2), jnp.int32),
            pltpu.VMEM((CHUNK, D // 2), jnp.int32),
            pltpu.VMEM((CHUNK, WIDTH), jnp.float32),
            pltpu.VMEM((CHUNK, WIDTH), jnp.float32),
            pltpu.VMEM((2 * KBLK, CHUNK), jnp.int32),
            pltpu.VMEM((2 * KBLK, CHUNK), jnp.int32),
            pltpu.VMEM((2 * KBLK, CHUNK), jnp.float32),
            pltpu.VMEM_SHARED((nodes_pad, WIDTH), jnp.float32),
            pltpu.SemaphoreType.DMA,
            pltpu.SemaphoreType.DMA,
            pltpu.SemaphoreType.DMA,
            pltpu.SemaphoreType.DMA,
            pltpu.SemaphoreType.DMA,
            pltpu.SemaphoreType.DMA,
        ],
    )(feats, srcp, dstp, wp, zrows)


def _tc_body(relu, p_ref, x_ref, wl_ref, wr_ref, b_ref, o_ref):
    p = p_ref[...]
    agg = p[0, :, :D] + p[1, :, :D]
    cnt = p[0, :, D:D + 1] + p[1, :, D:D + 1]
    inv = 1.0 / jnp.maximum(cnt, 1.0)
    y = jnp.dot(agg * inv, wl_ref[...], preferred_element_type=jnp.float32)
    y = y + jnp.dot(x_ref[...], wr_ref[...], preferred_element_type=jnp.float32)
    y = y + b_ref[...]
    if relu:
        y = jnp.maximum(y, 0.0)
    o_ref[...] = y


def _tc_sage(p, x, wl, wr, b, relu):
    n = x.shape[0]
    blk = 1000
    grid = n // blk
    return pl.pallas_call(
        functools.partial(_tc_body, relu),
        grid=(grid,),
        in_specs=[
            pl.BlockSpec((NC, blk, WIDTH), lambda i: (0, i, 0)),
            pl.BlockSpec((blk, D), lambda i: (i, 0)),
            pl.BlockSpec((D, D), lambda i: (0, 0)),
            pl.BlockSpec((D, D), lambda i: (0, 0)),
            pl.BlockSpec((1, D), lambda i: (0, 0)),
        ],
        out_specs=pl.BlockSpec((blk, D), lambda i: (i, 0)),
        out_shape=jax.ShapeDtypeStruct((n, D), jnp.float32),
    )(p, x, wl, wr, b)


def kernel(x, edge_index, edge_weight, node_type, Wl1, Wr1, b1, Wl2, Wr2, b2):
    del node_type
    n = x.shape[0]
    e = edge_weight.shape[0]
    nodes_pad = ((n + 1 + NS * 8 - 1) // (NS * 8)) * (NS * 8)
    grain = NW * CHUNK * KBLK
    ep = ((e + grain - 1) // grain) * grain

    src = edge_index[0].astype(jnp.int32)
    dst = edge_index[1].astype(jnp.int32)
    w = edge_weight.astype(jnp.float32)
    pad = ep - e
    srcp = jnp.concatenate([src, jnp.zeros((pad,), jnp.int32)])
    dstp = jnp.concatenate([dst, jnp.full((pad,), n, jnp.int32)])
    wp = jnp.concatenate([w, jnp.zeros((pad,), jnp.float32)])
    nchunk = ep // (NW * CHUNK)
    srcp = srcp.reshape(NW, nchunk, CHUNK)
    dstp = dstp.reshape(NW, nchunk, CHUNK)
    wp = wp.reshape(NW, nchunk, CHUNK)
    zrows = jnp.zeros((nodes_pad // NS, WIDTH), jnp.float32)
    perm = jnp.asarray(_PERM)
    wl1p = Wl1[perm, :]
    wl2p = Wl2[perm, :]
    b1r = b1.reshape(1, D)
    b2r = b2.reshape(1, D)

    xi = lax.bitcast_convert_type(
        x.astype(jnp.bfloat16).reshape(n, D // 2, 2), jnp.int32)
    p = _sc_agg(xi, srcp, dstp, wp, zrows, nodes_pad)
    h = _tc_sage(p, x, wl1p, Wr1, b1r, relu=True)
    hi32 = lax.bitcast_convert_type(
        h.astype(jnp.bfloat16).reshape(n, D // 2, 2), jnp.int32)
    q = _sc_agg(hi32, srcp, dstp, wp, zrows, nodes_pad)
    out = _tc_sage(q, h, wl2p, Wr2, b2r, relu=False)
    return out
